# Initial kernel scaffold; baseline (speedup 1.0000x reference)
import jax
import jax.numpy as jnp
from jax.experimental import pallas as pl
from jax.experimental.pallas import tpu as pltpu


def _add_body(x_ref, pe_ref, o_ref):
    o_ref[...] = x_ref[...] + pe_ref[:, None, :]


def kernel(x, pe):
    S, B, D = x.shape
    SB = 256
    grid = (S // SB,)
    return pl.pallas_call(
        _add_body,
        grid=grid,
        in_specs=[
            pl.BlockSpec((SB, B, D), lambda i: (i, 0, 0)),
            pl.BlockSpec((SB, D), lambda i: (i, 0)),
        ],
        out_specs=pl.BlockSpec((SB, B, D), lambda i: (i, 0, 0)),
        out_shape=jax.ShapeDtypeStruct((S, B, D), x.dtype),
    )(x, pe[:S])


# speedup vs baseline: 2.1165x; 2.1165x over previous
import jax
import jax.numpy as jnp
from jax.experimental import pallas as pl
from jax.experimental.pallas import tpu as pltpu


def _add_body(x_ref, pe_ref, o_ref):
    o_ref[...] = x_ref[...] + pe_ref[...][:, None, :]


def kernel(x, pe):
    S, B, D = x.shape
    SB = 256
    grid = (S // SB,)
    return pl.pallas_call(
        _add_body,
        grid=grid,
        in_specs=[
            pl.BlockSpec((SB, B, D), lambda i: (i, 0, 0)),
            pl.BlockSpec((SB, D), lambda i: (i, 0)),
        ],
        out_specs=pl.BlockSpec((SB, B, D), lambda i: (i, 0, 0)),
        out_shape=jax.ShapeDtypeStruct((S, B, D), x.dtype),
    )(x, pe[:S])
